# baseline (speedup 1.0000x reference)
import functools

import jax
import jax.numpy as jnp
from jax import lax
from jax.experimental import pallas as pl
from jax.experimental.pallas import tpu as pltpu
from jax.experimental.pallas import tpu_sc as plsc

_NUM_WORKERS = 32
_LANES = 16


def _make_mask_kernel(v_pad: int, g_pad: int):
    chunk = v_pad // _NUM_WORKERS
    mesh = plsc.VectorSubcoreMesh(core_axis_name="c", subcore_axis_name="s")

    @functools.partial(
        pl.kernel,
        mesh=mesh,
        out_type=jax.ShapeDtypeStruct((v_pad,), jnp.float32),
        scratch_types=[
            pltpu.VMEM((g_pad,), jnp.int32),
            pltpu.VMEM((chunk,), jnp.float32),
        ],
        compiler_params=pltpu.CompilerParams(needs_layout_passes=False),
    )
    def mask_kernel(grammar_hbm, mask_hbm, idx_v, buf_v):
        c = lax.axis_index("c")
        s = lax.axis_index("s")
        wid = s * 2 + c
        base = wid * chunk

        zeros16 = jnp.zeros((_LANES,), jnp.float32)

        def zero_body(i, carry):
            buf_v[pl.ds(i * _LANES, _LANES)] = zeros16
            return carry

        lax.fori_loop(0, chunk // _LANES, zero_body, 0)

        pltpu.sync_copy(grammar_hbm, idx_v)

        ones16 = jnp.ones((_LANES,), jnp.float32)

        def scatter_body(j, carry):
            idx = idx_v[pl.ds(j * _LANES, _LANES)]
            m = (idx >= base) & (idx < base + chunk)
            local = jnp.where(m, idx - base, 0)
            plsc.store_scatter(buf_v, [local], ones16, mask=m)
            return carry

        lax.fori_loop(0, g_pad // _LANES, scatter_body, 0)

        pltpu.sync_copy(buf_v, mask_hbm.at[pl.ds(base, chunk)])

    return mask_kernel


def _blend_body(state_ref, w_ref, b_ref, x_ref, m_ref, o_ref, arfa_ref):
    @pl.when(pl.program_id(0) == 0)
    def _():
        z = jnp.sum(state_ref[...] * w_ref[...], axis=1, keepdims=True) + b_ref[...]
        arfa_ref[...] = jax.nn.sigmoid(z)

    arfa = arfa_ref[...]
    o_ref[...] = x_ref[...] * (1.0 + m_ref[...] * (arfa - 1.0))


def kernel(output, state, grammar, W, b):
    B, _, V = output.shape
    H = state.shape[-1]
    G = grammar.shape[0]

    vblk = 2048
    n_blocks = -(-V // vblk)
    v_pad = n_blocks * vblk
    assert v_pad % (_NUM_WORKERS * _LANES) == 0
    g_pad = -(-G // _LANES) * _LANES

    gpad = jnp.concatenate(
        [grammar, jnp.full((g_pad - G,), -1, jnp.int32)]
    )

    mask = _make_mask_kernel(v_pad, g_pad)(gpad)
    mask2d = mask.reshape(1, v_pad)

    x2d = output.reshape(B, V)
    state2d = state.reshape(B, H)
    b2d = b.reshape(1, 1)

    out2d = pl.pallas_call(
        _blend_body,
        grid=(n_blocks,),
        in_specs=[
            pl.BlockSpec((B, H), lambda i: (0, 0)),
            pl.BlockSpec((1, H), lambda i: (0, 0)),
            pl.BlockSpec((1, 1), lambda i: (0, 0)),
            pl.BlockSpec((B, vblk), lambda i: (0, i)),
            pl.BlockSpec((1, vblk), lambda i: (0, i)),
        ],
        out_specs=pl.BlockSpec((B, vblk), lambda i: (0, i)),
        out_shape=jax.ShapeDtypeStruct((B, V), jnp.float32),
        scratch_shapes=[pltpu.VMEM((B, 1), jnp.float32)],
    )(state2d, W, b2d, x2d, mask2d)

    return out2d.reshape(B, 1, V)


# speedup vs baseline: 1.8777x; 1.8777x over previous
import functools

import jax
import jax.numpy as jnp
from jax import lax
from jax.experimental import pallas as pl
from jax.experimental.pallas import tpu as pltpu
from jax.experimental.pallas import tpu_sc as plsc

_NUM_WORKERS = 32
_LANES = 16


def _make_mask_kernel(rows: int, g_rows: int):
    rows_per_w = rows // _NUM_WORKERS
    chunk = rows_per_w * 128
    mesh = plsc.VectorSubcoreMesh(core_axis_name="c", subcore_axis_name="s")

    @functools.partial(
        pl.kernel,
        mesh=mesh,
        out_type=jax.ShapeDtypeStruct((rows, 128), jnp.float32),
        scratch_types=[
            pltpu.VMEM((g_rows, 128), jnp.int32),
            pltpu.VMEM((rows_per_w, 128), jnp.float32),
        ],
        compiler_params=pltpu.CompilerParams(needs_layout_passes=False),
    )
    def mask_kernel(grammar_hbm, mask_hbm, idx_v, buf_v):
        c = lax.axis_index("c")
        s = lax.axis_index("s")
        wid = s * 2 + c
        base = wid * chunk

        zeros16 = jnp.zeros((_LANES,), jnp.float32)

        def zero_body(i, carry):
            buf_v[i // 8, pl.ds((i % 8) * _LANES, _LANES)] = zeros16
            return carry

        lax.fori_loop(0, rows_per_w * 8, zero_body, 0)

        pltpu.sync_copy(grammar_hbm, idx_v)

        ones16 = jnp.ones((_LANES,), jnp.float32)

        def scatter_body(j, carry):
            idx = idx_v[j // 8, pl.ds((j % 8) * _LANES, _LANES)]
            m = (idx >= base) & (idx < base + chunk)
            local = jnp.where(m, idx - base, 0)
            row = lax.shift_right_logical(local, 7)
            col = lax.bitwise_and(local, 127)
            plsc.store_scatter(buf_v, [row, col], ones16, mask=m)
            return carry

        lax.fori_loop(0, g_rows * 8, scatter_body, 0)

        pltpu.sync_copy(buf_v, mask_hbm.at[pl.ds(wid * rows_per_w, rows_per_w), :])

    return mask_kernel


def _blend_body(state_ref, w_ref, b_ref, x_ref, m_ref, o_ref, arfa_ref):
    @pl.when(pl.program_id(0) == 0)
    def _():
        z = lax.dot_general(
            w_ref[...],
            state_ref[...],
            (((1,), (1,)), ((), ())),
            preferred_element_type=jnp.float32,
        )
        arfa_ref[...] = jax.nn.sigmoid(z + b_ref[...])

    gate = arfa_ref[...] - 1.0
    m_t = m_ref[...].T
    for t in range(m_ref.shape[0]):
        m_col = m_t[:, t : t + 1]
        sl = slice(t * 128, (t + 1) * 128)
        o_ref[sl, :] = x_ref[sl, :] * (1.0 + m_col * gate)


def kernel(output, state, grammar, W, b):
    B, _, V = output.shape
    H = state.shape[-1]
    G = grammar.shape[0]

    vblk = 2048
    tiles_per_blk = vblk // 128
    n_blocks = -(-V // vblk)

    rows = -(-(n_blocks * tiles_per_blk) // (_NUM_WORKERS * 8)) * (_NUM_WORKERS * 8)
    g_rows = -(-G // 128)

    gpad = jnp.concatenate(
        [grammar, jnp.full((g_rows * 128 - G,), -1, jnp.int32)]
    ).reshape(g_rows, 128)

    mask = _make_mask_kernel(rows, g_rows)(gpad)

    xt = jnp.transpose(output, (1, 2, 0)).reshape(V, B)
    state2d = state.reshape(B, H)
    b2d = b.reshape(1, 1)

    out_t = pl.pallas_call(
        _blend_body,
        grid=(n_blocks,),
        in_specs=[
            pl.BlockSpec((B, H), lambda i: (0, 0)),
            pl.BlockSpec((1, H), lambda i: (0, 0)),
            pl.BlockSpec((1, 1), lambda i: (0, 0)),
            pl.BlockSpec((vblk, B), lambda i: (i, 0)),
            pl.BlockSpec((tiles_per_blk, 128), lambda i: (i, 0)),
        ],
        out_specs=pl.BlockSpec((vblk, B), lambda i: (i, 0)),
        out_shape=jax.ShapeDtypeStruct((V, B), jnp.float32),
        scratch_shapes=[pltpu.VMEM((1, B), jnp.float32)],
    )(state2d, W, b2d, xt, mask)

    return jnp.transpose(out_t.reshape(1, V, B), (2, 0, 1))


# speedup vs baseline: 2.3277x; 1.2397x over previous
import functools

import jax
import jax.numpy as jnp
from jax import lax
from jax.experimental import pallas as pl
from jax.experimental.pallas import tpu as pltpu
from jax.experimental.pallas import tpu_sc as plsc

_NUM_WORKERS = 32
_LANES = 16


def _make_mask_kernel(rows: int, g_rows: int):
    rows_per_w = rows // _NUM_WORKERS
    chunk = rows_per_w * 128
    mesh = plsc.VectorSubcoreMesh(core_axis_name="c", subcore_axis_name="s")

    @functools.partial(
        pl.kernel,
        mesh=mesh,
        out_type=jax.ShapeDtypeStruct((rows, 128), jnp.float32),
        scratch_types=[
            pltpu.VMEM((g_rows, 128), jnp.int32),
            pltpu.VMEM((rows_per_w, 128), jnp.float32),
        ],
        compiler_params=pltpu.CompilerParams(needs_layout_passes=False),
    )
    def mask_kernel(grammar_hbm, mask_hbm, idx_v, buf_v):
        c = lax.axis_index("c")
        s = lax.axis_index("s")
        wid = s * 2 + c
        base = wid * chunk

        zeros16 = jnp.zeros((_LANES,), jnp.float32)

        def zero_body(i, carry):
            buf_v[i // 8, pl.ds((i % 8) * _LANES, _LANES)] = zeros16
            return carry

        lax.fori_loop(0, rows_per_w * 8, zero_body, 0)

        pltpu.sync_copy(grammar_hbm, idx_v)

        ones16 = jnp.ones((_LANES,), jnp.float32)

        def scatter_body(j, carry):
            idx = idx_v[j // 8, pl.ds((j % 8) * _LANES, _LANES)]
            m = (idx >= base) & (idx < base + chunk)
            local = jnp.where(m, idx - base, 0)
            row = lax.shift_right_logical(local, 7)
            col = lax.bitwise_and(local, 127)
            plsc.store_scatter(buf_v, [row, col], ones16, mask=m)
            return carry

        lax.fori_loop(0, g_rows * 8, scatter_body, 0)

        pltpu.sync_copy(buf_v, mask_hbm.at[pl.ds(wid * rows_per_w, rows_per_w), :])

    return mask_kernel


def _blend_body(state_ref, w_ref, b_ref, x_ref, m_ref, o_ref, arfa_ref):
    @pl.when(pl.program_id(0) == 0)
    def _():
        z = lax.dot_general(
            w_ref[...],
            state_ref[...],
            (((1,), (1,)), ((), ())),
            preferred_element_type=jnp.float32,
        )
        arfa_ref[...] = jax.nn.sigmoid(z + b_ref[...])

    gate = arfa_ref[...] - 1.0
    m_t = m_ref[...].T
    for t in range(m_ref.shape[0]):
        m_col = m_t[:, t : t + 1]
        sl = slice(t * 128, (t + 1) * 128)
        o_ref[sl, :] = x_ref[sl, :] * (1.0 + m_col * gate)


def kernel(output, state, grammar, W, b):
    B, _, V = output.shape
    H = state.shape[-1]
    G = grammar.shape[0]

    vblk = 4096
    tiles_per_blk = vblk // 128
    n_blocks = -(-V // vblk)

    rows = -(-(n_blocks * tiles_per_blk) // (_NUM_WORKERS * 8)) * (_NUM_WORKERS * 8)
    g_rows = -(-G // 128)

    gpad = jnp.concatenate(
        [grammar, jnp.full((g_rows * 128 - G,), -1, jnp.int32)]
    ).reshape(g_rows, 128)

    mask = _make_mask_kernel(rows, g_rows)(gpad)

    xt = jnp.transpose(output, (1, 2, 0)).reshape(V, B)
    state2d = state.reshape(B, H)
    b2d = b.reshape(1, 1)

    out_t = pl.pallas_call(
        _blend_body,
        grid=(n_blocks,),
        in_specs=[
            pl.BlockSpec((B, H), lambda i: (0, 0)),
            pl.BlockSpec((1, H), lambda i: (0, 0)),
            pl.BlockSpec((1, 1), lambda i: (0, 0)),
            pl.BlockSpec((vblk, B), lambda i: (i, 0)),
            pl.BlockSpec((tiles_per_blk, 128), lambda i: (i, 0)),
        ],
        out_specs=pl.BlockSpec((vblk, B), lambda i: (i, 0)),
        out_shape=jax.ShapeDtypeStruct((V, B), jnp.float32),
        scratch_shapes=[pltpu.VMEM((1, B), jnp.float32)],
    )(state2d, W, b2d, xt, mask)

    return jnp.transpose(out_t.reshape(1, V, B), (2, 0, 1))


# speedup vs baseline: 2.5658x; 1.1023x over previous
import functools

import jax
import jax.numpy as jnp
from jax import lax
from jax.experimental import pallas as pl
from jax.experimental.pallas import tpu as pltpu
from jax.experimental.pallas import tpu_sc as plsc

_NUM_WORKERS = 32
_LANES = 16


def _make_mask_kernel(rows: int, g_rows: int):
    rows_per_w = rows // _NUM_WORKERS
    chunk = rows_per_w * 128
    mesh = plsc.VectorSubcoreMesh(core_axis_name="c", subcore_axis_name="s")

    @functools.partial(
        pl.kernel,
        mesh=mesh,
        out_type=jax.ShapeDtypeStruct((rows, 128), jnp.float32),
        scratch_types=[
            pltpu.VMEM((g_rows, 128), jnp.int32),
            pltpu.VMEM((rows_per_w, 128), jnp.float32),
        ],
        compiler_params=pltpu.CompilerParams(needs_layout_passes=False),
    )
    def mask_kernel(grammar_hbm, mask_hbm, idx_v, buf_v):
        c = lax.axis_index("c")
        s = lax.axis_index("s")
        wid = s * 2 + c
        base = wid * chunk

        zeros16 = jnp.zeros((_LANES,), jnp.float32)

        def zero_body(i, carry):
            buf_v[i // 8, pl.ds((i % 8) * _LANES, _LANES)] = zeros16
            return carry

        lax.fori_loop(0, rows_per_w * 8, zero_body, 0)

        pltpu.sync_copy(grammar_hbm, idx_v)

        ones16 = jnp.ones((_LANES,), jnp.float32)

        def scatter_body(j, carry):
            idx = idx_v[j // 8, pl.ds((j % 8) * _LANES, _LANES)]
            m = (idx >= base) & (idx < base + chunk)
            local = jnp.where(m, idx - base, 0)
            row = lax.shift_right_logical(local, 7)
            col = lax.bitwise_and(local, 127)
            plsc.store_scatter(buf_v, [row, col], ones16, mask=m)
            return carry

        lax.fori_loop(0, g_rows * 8, scatter_body, 0)

        pltpu.sync_copy(buf_v, mask_hbm.at[pl.ds(wid * rows_per_w, rows_per_w), :])

    return mask_kernel


def _blend_body(state_ref, w_ref, b_ref, x_ref, m_ref, o_ref, arfa_ref):
    @pl.when(pl.program_id(0) == 0)
    def _():
        z = lax.dot_general(
            w_ref[...],
            state_ref[...],
            (((1,), (1,)), ((), ())),
            preferred_element_type=jnp.float32,
        )
        arfa_ref[...] = jax.nn.sigmoid(z + b_ref[...])

    gate = arfa_ref[...] - 1.0
    m_t = m_ref[...].T
    for t in range(m_ref.shape[0]):
        m_col = m_t[:, t : t + 1]
        sl = slice(t * 128, (t + 1) * 128)
        o_ref[sl, :] = x_ref[sl, :] * (1.0 + m_col * gate)


def kernel(output, state, grammar, W, b):
    B, _, V = output.shape
    H = state.shape[-1]
    G = grammar.shape[0]

    vblk = 8192
    tiles_per_blk = vblk // 128
    n_blocks = -(-V // vblk)

    rows = -(-(n_blocks * tiles_per_blk) // (_NUM_WORKERS * 8)) * (_NUM_WORKERS * 8)
    g_rows = -(-G // 128)

    gpad = jnp.concatenate(
        [grammar, jnp.full((g_rows * 128 - G,), -1, jnp.int32)]
    ).reshape(g_rows, 128)

    mask = _make_mask_kernel(rows, g_rows)(gpad)

    xt = jnp.transpose(output, (1, 2, 0)).reshape(V, B)
    state2d = state.reshape(B, H)
    b2d = b.reshape(1, 1)

    out_t = pl.pallas_call(
        _blend_body,
        grid=(n_blocks,),
        in_specs=[
            pl.BlockSpec((B, H), lambda i: (0, 0)),
            pl.BlockSpec((1, H), lambda i: (0, 0)),
            pl.BlockSpec((1, 1), lambda i: (0, 0)),
            pl.BlockSpec((vblk, B), lambda i: (i, 0)),
            pl.BlockSpec((tiles_per_blk, 128), lambda i: (i, 0)),
        ],
        out_specs=pl.BlockSpec((vblk, B), lambda i: (i, 0)),
        out_shape=jax.ShapeDtypeStruct((V, B), jnp.float32),
        scratch_shapes=[pltpu.VMEM((1, B), jnp.float32)],
    )(state2d, W, b2d, xt, mask)

    return jnp.transpose(out_t.reshape(1, V, B), (2, 0, 1))


# speedup vs baseline: 2.5873x; 1.0084x over previous
import functools

import jax
import jax.numpy as jnp
from jax import lax
from jax.experimental import pallas as pl
from jax.experimental.pallas import tpu as pltpu
from jax.experimental.pallas import tpu_sc as plsc

_NUM_WORKERS = 32
_LANES = 16


def _make_mask_kernel(rows: int, g_rows: int):
    rows_per_w = rows // _NUM_WORKERS
    chunk = rows_per_w * 128
    mesh = plsc.VectorSubcoreMesh(core_axis_name="c", subcore_axis_name="s")

    @functools.partial(
        pl.kernel,
        mesh=mesh,
        out_type=jax.ShapeDtypeStruct((rows, 128), jnp.float32),
        scratch_types=[
            pltpu.VMEM((g_rows, 128), jnp.int32),
            pltpu.VMEM((rows_per_w, 128), jnp.float32),
        ],
        compiler_params=pltpu.CompilerParams(needs_layout_passes=False),
    )
    def mask_kernel(grammar_hbm, mask_hbm, idx_v, buf_v):
        c = lax.axis_index("c")
        s = lax.axis_index("s")
        wid = s * 2 + c
        base = wid * chunk

        zeros16 = jnp.zeros((_LANES,), jnp.float32)

        def zero_body(i, carry):
            buf_v[i // 8, pl.ds((i % 8) * _LANES, _LANES)] = zeros16
            return carry

        lax.fori_loop(0, rows_per_w * 8, zero_body, 0)

        pltpu.sync_copy(grammar_hbm, idx_v)

        ones16 = jnp.ones((_LANES,), jnp.float32)

        def scatter_body(j, carry):
            idx = idx_v[j // 8, pl.ds((j % 8) * _LANES, _LANES)]
            m = (idx >= base) & (idx < base + chunk)
            local = jnp.where(m, idx - base, 0)
            row = lax.shift_right_logical(local, 7)
            col = lax.bitwise_and(local, 127)
            plsc.store_scatter(buf_v, [row, col], ones16, mask=m)
            return carry

        lax.fori_loop(0, g_rows * 8, scatter_body, 0)

        pltpu.sync_copy(buf_v, mask_hbm.at[pl.ds(wid * rows_per_w, rows_per_w), :])

    return mask_kernel


def _blend_body(state_ref, w_ref, b_ref, x_ref, m_ref, o_ref, arfa_ref):
    @pl.when(pl.program_id(0) == 0)
    def _():
        z = lax.dot_general(
            w_ref[...],
            state_ref[...],
            (((1,), (1,)), ((), ())),
            preferred_element_type=jnp.float32,
        )
        arfa_ref[...] = jax.nn.sigmoid(z + b_ref[...])

    gate = arfa_ref[...] - 1.0
    m_t = m_ref[...].T
    for t in range(m_ref.shape[0]):
        m_col = m_t[:, t : t + 1]
        sl = slice(t * 128, (t + 1) * 128)
        o_ref[sl, :] = x_ref[sl, :] * (1.0 + m_col * gate)


def kernel(output, state, grammar, W, b):
    B, _, V = output.shape
    H = state.shape[-1]
    G = grammar.shape[0]

    vblk = 16384
    tiles_per_blk = vblk // 128
    n_blocks = -(-V // vblk)

    rows = -(-(n_blocks * tiles_per_blk) // (_NUM_WORKERS * 8)) * (_NUM_WORKERS * 8)
    g_rows = -(-G // 128)

    gpad = jnp.concatenate(
        [grammar, jnp.full((g_rows * 128 - G,), -1, jnp.int32)]
    ).reshape(g_rows, 128)

    mask = _make_mask_kernel(rows, g_rows)(gpad)

    xt = jnp.transpose(output, (1, 2, 0)).reshape(V, B)
    state2d = state.reshape(B, H)
    b2d = b.reshape(1, 1)

    out_t = pl.pallas_call(
        _blend_body,
        grid=(n_blocks,),
        in_specs=[
            pl.BlockSpec((B, H), lambda i: (0, 0)),
            pl.BlockSpec((1, H), lambda i: (0, 0)),
            pl.BlockSpec((1, 1), lambda i: (0, 0)),
            pl.BlockSpec((vblk, B), lambda i: (i, 0)),
            pl.BlockSpec((tiles_per_blk, 128), lambda i: (i, 0)),
        ],
        out_specs=pl.BlockSpec((vblk, B), lambda i: (i, 0)),
        out_shape=jax.ShapeDtypeStruct((V, B), jnp.float32),
        scratch_shapes=[pltpu.VMEM((1, B), jnp.float32)],
    )(state2d, W, b2d, xt, mask)

    return jnp.transpose(out_t.reshape(1, V, B), (2, 0, 1))


# speedup vs baseline: 2.5984x; 1.0043x over previous
import functools

import jax
import jax.numpy as jnp
from jax import lax
from jax.experimental import pallas as pl
from jax.experimental.pallas import tpu as pltpu
from jax.experimental.pallas import tpu_sc as plsc

_NUM_WORKERS = 32
_LANES = 16


def _make_mask_kernel(rows: int, g_rows: int):
    rows_per_w = rows // _NUM_WORKERS
    chunk = rows_per_w * 128
    mesh = plsc.VectorSubcoreMesh(core_axis_name="c", subcore_axis_name="s")

    @functools.partial(
        pl.kernel,
        mesh=mesh,
        out_type=jax.ShapeDtypeStruct((rows, 128), jnp.float32),
        scratch_types=[
            pltpu.VMEM((g_rows, 128), jnp.int32),
            pltpu.VMEM((rows_per_w, 128), jnp.float32),
        ],
        compiler_params=pltpu.CompilerParams(needs_layout_passes=False),
    )
    def mask_kernel(grammar_hbm, mask_hbm, idx_v, buf_v):
        c = lax.axis_index("c")
        s = lax.axis_index("s")
        wid = s * 2 + c
        base = wid * chunk

        zeros16 = jnp.zeros((_LANES,), jnp.float32)

        def zero_body(i, carry):
            buf_v[i // 8, pl.ds((i % 8) * _LANES, _LANES)] = zeros16
            return carry

        lax.fori_loop(0, rows_per_w * 8, zero_body, 0)

        pltpu.sync_copy(grammar_hbm, idx_v)

        ones16 = jnp.ones((_LANES,), jnp.float32)

        def scatter_body(j, carry):
            idx = idx_v[j // 8, pl.ds((j % 8) * _LANES, _LANES)]
            m = (idx >= base) & (idx < base + chunk)
            local = jnp.where(m, idx - base, 0)
            row = lax.shift_right_logical(local, 7)
            col = lax.bitwise_and(local, 127)
            plsc.store_scatter(buf_v, [row, col], ones16, mask=m)
            return carry

        lax.fori_loop(0, g_rows * 8, scatter_body, 0)

        pltpu.sync_copy(buf_v, mask_hbm.at[pl.ds(wid * rows_per_w, rows_per_w), :])

    return mask_kernel


def _blend_body(state_ref, w_ref, b_ref, x_ref, m_ref, o_ref, arfa_ref):
    @pl.when(pl.program_id(0) == 0)
    def _():
        z = lax.dot_general(
            w_ref[...],
            state_ref[...],
            (((1,), (1,)), ((), ())),
            preferred_element_type=jnp.float32,
        )
        arfa_ref[...] = jax.nn.sigmoid(z + b_ref[...])

    gate = arfa_ref[...] - 1.0
    m_t = m_ref[...].T
    for t in range(m_ref.shape[0]):
        m_col = m_t[:, t : t + 1]
        sl = slice(t * 128, (t + 1) * 128)
        o_ref[sl, :] = x_ref[sl, :] * (1.0 + m_col * gate)


def kernel(output, state, grammar, W, b):
    B, _, V = output.shape
    H = state.shape[-1]
    G = grammar.shape[0]

    vblk = 14336
    tiles_per_blk = vblk // 128
    n_blocks = -(-V // vblk)

    rows = -(-(n_blocks * tiles_per_blk) // (_NUM_WORKERS * 8)) * (_NUM_WORKERS * 8)
    g_rows = -(-G // 128)

    gpad = jnp.concatenate(
        [grammar, jnp.full((g_rows * 128 - G,), -1, jnp.int32)]
    ).reshape(g_rows, 128)

    mask = _make_mask_kernel(rows, g_rows)(gpad)

    xt = jnp.transpose(output, (1, 2, 0)).reshape(V, B)
    state2d = state.reshape(B, H)
    b2d = b.reshape(1, 1)

    out_t = pl.pallas_call(
        _blend_body,
        grid=(n_blocks,),
        in_specs=[
            pl.BlockSpec((B, H), lambda i: (0, 0)),
            pl.BlockSpec((1, H), lambda i: (0, 0)),
            pl.BlockSpec((1, 1), lambda i: (0, 0)),
            pl.BlockSpec((vblk, B), lambda i: (i, 0)),
            pl.BlockSpec((tiles_per_blk, 128), lambda i: (i, 0)),
        ],
        out_specs=pl.BlockSpec((vblk, B), lambda i: (i, 0)),
        out_shape=jax.ShapeDtypeStruct((V, B), jnp.float32),
        scratch_shapes=[pltpu.VMEM((1, B), jnp.float32)],
    )(state2d, W, b2d, xt, mask)

    return jnp.transpose(out_t.reshape(1, V, B), (2, 0, 1))


# speedup vs baseline: 2.6569x; 1.0225x over previous
import functools

import jax
import jax.numpy as jnp
from jax import lax
from jax.experimental import pallas as pl
from jax.experimental.pallas import tpu as pltpu
from jax.experimental.pallas import tpu_sc as plsc

_NUM_WORKERS = 32
_LANES = 16


def _make_mask_kernel(rows: int, g_rows: int):
    rows_per_w = rows // _NUM_WORKERS
    chunk = rows_per_w * 128
    mesh = plsc.VectorSubcoreMesh(core_axis_name="c", subcore_axis_name="s")

    @functools.partial(
        pl.kernel,
        mesh=mesh,
        out_type=jax.ShapeDtypeStruct((rows, 128), jnp.float32),
        scratch_types=[
            pltpu.VMEM((g_rows, 128), jnp.int32),
            pltpu.VMEM((rows_per_w, 128), jnp.float32),
            pltpu.SemaphoreType.DMA,
        ],
        compiler_params=pltpu.CompilerParams(needs_layout_passes=False),
    )
    def mask_kernel(grammar_hbm, mask_hbm, idx_v, buf_v, sem):
        c = lax.axis_index("c")
        s = lax.axis_index("s")
        wid = s * 2 + c
        base = wid * chunk

        gcopy = pltpu.async_copy(grammar_hbm, idx_v, sem)

        zeros16 = jnp.zeros((_LANES,), jnp.float32)

        def zero_body(i, carry):
            buf_v[i // 8, pl.ds((i % 8) * _LANES, _LANES)] = zeros16
            return carry

        lax.fori_loop(0, rows_per_w * 8, zero_body, 0, unroll=8)

        gcopy.wait()

        ones16 = jnp.ones((_LANES,), jnp.float32)

        def scatter_body(j, carry):
            idx = idx_v[j // 8, pl.ds((j % 8) * _LANES, _LANES)]
            m = (idx >= base) & (idx < base + chunk)
            local = jnp.where(m, idx - base, 0)
            row = lax.shift_right_logical(local, 7)
            col = lax.bitwise_and(local, 127)
            plsc.store_scatter(buf_v, [row, col], ones16, mask=m)
            return carry

        lax.fori_loop(0, g_rows * 8, scatter_body, 0, unroll=8)

        pltpu.sync_copy(buf_v, mask_hbm.at[pl.ds(wid * rows_per_w, rows_per_w), :])

    return mask_kernel


def _blend_body(state_ref, w_ref, b_ref, x_ref, m_ref, o_ref, arfa_ref):
    @pl.when(pl.program_id(0) == 0)
    def _():
        z = lax.dot_general(
            w_ref[...],
            state_ref[...],
            (((1,), (1,)), ((), ())),
            preferred_element_type=jnp.float32,
        )
        arfa_ref[...] = jax.nn.sigmoid(z + b_ref[...])

    gate = arfa_ref[...] - 1.0
    m_t = m_ref[...].T
    for t in range(m_ref.shape[0]):
        m_col = m_t[:, t : t + 1]
        sl = slice(t * 128, (t + 1) * 128)
        o_ref[sl, :] = x_ref[sl, :] * (1.0 + m_col * gate)


def kernel(output, state, grammar, W, b):
    B, _, V = output.shape
    H = state.shape[-1]
    G = grammar.shape[0]

    vblk = 14336
    tiles_per_blk = vblk // 128
    n_blocks = -(-V // vblk)

    rows = -(-(n_blocks * tiles_per_blk) // (_NUM_WORKERS * 8)) * (_NUM_WORKERS * 8)
    g_rows = -(-G // 128)

    gpad = jnp.concatenate(
        [grammar, jnp.full((g_rows * 128 - G,), -1, jnp.int32)]
    ).reshape(g_rows, 128)

    mask = _make_mask_kernel(rows, g_rows)(gpad)

    xt = jnp.transpose(output, (1, 2, 0)).reshape(V, B)
    state2d = state.reshape(B, H)
    b2d = b.reshape(1, 1)

    out_t = pl.pallas_call(
        _blend_body,
        grid=(n_blocks,),
        in_specs=[
            pl.BlockSpec((B, H), lambda i: (0, 0)),
            pl.BlockSpec((1, H), lambda i: (0, 0)),
            pl.BlockSpec((1, 1), lambda i: (0, 0)),
            pl.BlockSpec((vblk, B), lambda i: (i, 0)),
            pl.BlockSpec((tiles_per_blk, 128), lambda i: (i, 0)),
        ],
        out_specs=pl.BlockSpec((vblk, B), lambda i: (i, 0)),
        out_shape=jax.ShapeDtypeStruct((V, B), jnp.float32),
        scratch_shapes=[pltpu.VMEM((1, B), jnp.float32)],
    )(state2d, W, b2d, xt, mask)

    return jnp.transpose(out_t.reshape(1, V, B), (2, 0, 1))
